# Initial kernel scaffold; baseline (speedup 1.0000x reference)
import jax
import jax.numpy as jnp
from jax.experimental import pallas as pl


def kernel(x, adj, W1, b1, W2, b2, Wl1, bl1, Wl2, bl2):
    raise NotImplementedError("write your pallas kernel here")


# speedup vs baseline: 11.7253x; 11.7253x over previous
import functools

import jax
import jax.numpy as jnp
from jax import lax
from jax.experimental import pallas as pl
from jax.experimental.pallas import tpu as pltpu
from jax.experimental.pallas import tpu_sc as plsc

_N = 10000
_E = 320000
_D = 128
_NP = 10240
_NC, _NS = 2, 16
_NW = _NC * _NS
_EPW = _E // _NW
_K = 80
_NCH = _EPW // _K
_RPT = _NP // _NS
_B = 512
_GRID = _NP // _B

_mesh = plsc.VectorSubcoreMesh(core_axis_name="c", subcore_axis_name="s")


@functools.partial(
    pl.kernel,
    out_type=jax.ShapeDtypeStruct((_NW, _NP), jnp.float32),
    mesh=_mesh,
    scratch_types=[
        pltpu.VMEM((_EPW,), jnp.int32),
        pltpu.VMEM((_NP,), jnp.float32),
    ],
    compiler_params=pltpu.CompilerParams(needs_layout_passes=False),
)
def _sc_degree(dst_hbm, degp_hbm, dbuf, hist):
    wid = lax.axis_index("c") * _NS + lax.axis_index("s")
    pltpu.sync_copy(dst_hbm.at[pl.ds(wid * _EPW, _EPW)], dbuf)
    zeros = jnp.zeros((16,), jnp.float32)

    def zbody(i, _):
        hist[pl.ds(i * 16, 16)] = zeros
        return 0

    lax.fori_loop(0, _NP // 16, zbody, 0)
    ones = jnp.ones((16,), jnp.float32)

    def body(i, _):
        idx = dbuf[pl.ds(i * 16, 16)]
        plsc.addupdate_scatter(hist, [idx], ones)
        return 0

    lax.fori_loop(0, _EPW // 16, body, 0)
    pltpu.sync_copy(hist, degp_hbm.at[wid])


@functools.partial(
    pl.kernel,
    out_type=jax.ShapeDtypeStruct((_NC, _NP, _D), jnp.float32),
    mesh=_mesh,
    scratch_types=[
        pltpu.VMEM((_K,), jnp.int32),
        pltpu.VMEM((_K,), jnp.int32),
        pltpu.VMEM((_K, _D), jnp.float32),
        pltpu.VMEM_SHARED((_NP, _D), jnp.float32),
        pltpu.SemaphoreType.DMA,
    ],
)
def _sc_pass(src_hbm, dst_hbm, xt_hbm, aggp_hbm, sidx, didx, rows, acc, sem):
    cid = lax.axis_index("c")
    sid = lax.axis_index("s")
    wid = cid * _NS + sid
    zeros = jnp.zeros((16,), jnp.float32)

    def zb(i, _):
        rows[i // 8, pl.ds((i % 8) * 16, 16)] = zeros
        return 0

    lax.fori_loop(0, _K * 8, zb, 0)

    def za(i, _):
        pltpu.sync_copy(rows, acc.at[pl.ds(sid * _RPT + i * _K, _K)])
        return 0

    lax.fori_loop(0, _RPT // _K, za, 0)
    plsc.subcore_barrier()

    base = wid * _EPW

    def body(j, _):
        off = base + j * _K
        pltpu.sync_copy(src_hbm.at[pl.ds(off, _K)], sidx)
        pltpu.sync_copy(dst_hbm.at[pl.ds(off, _K)], didx)
        pltpu.async_copy(xt_hbm.at[sidx], rows, sem).wait()
        pltpu.sync_copy(rows, acc.at[didx], add=True)
        return 0

    lax.fori_loop(0, _NCH, body, 0)
    plsc.subcore_barrier()
    pltpu.sync_copy(
        acc.at[pl.ds(sid * _RPT, _RPT)],
        aggp_hbm.at[cid, pl.ds(sid * _RPT, _RPT)],
    )


def _dinv(degp):
    ones = jnp.ones((_NW, 1), jnp.float32)
    deg = lax.dot_general(degp, ones, (((0,), (0,)), ((), ())))
    return lax.rsqrt(jnp.maximum(deg, 1.0))


def _tc_pre_body(x_ref, w1_ref, degp_ref, xt1_ref):
    xw = jnp.dot(x_ref[...], w1_ref[...], preferred_element_type=jnp.float32)
    xt1_ref[...] = xw * _dinv(degp_ref[...])


def _tc_mid_body(aggp_ref, degp_ref, b1_ref, xt2_ref):
    di = _dinv(degp_ref[...])
    h1 = jnp.maximum(di * (aggp_ref[0] + aggp_ref[1]) + b1_ref[...], 0.0)
    xt2_ref[...] = di * h1


def _tc_final_body(aggp_ref, degp_ref, w2_ref, wl1_ref, b2_ref, bl1_ref,
                   wl2_ref, bl2_ref, out_ref, acc, w25, b25):
    i = pl.program_id(0)

    @pl.when(i == 0)
    def _():
        w25[...] = jnp.dot(w2_ref[...], wl1_ref[...],
                           preferred_element_type=jnp.float32)
        b25[...] = jnp.dot(b2_ref[...], wl1_ref[...],
                           preferred_element_type=jnp.float32) + bl1_ref[...]
        acc[...] = jnp.zeros((1, _D), jnp.float32)

    a2 = _dinv(degp_ref[...]) * (aggp_ref[0] + aggp_ref[1])
    h = jnp.maximum(
        jnp.dot(a2, w25[...], preferred_element_type=jnp.float32) + b25[...],
        0.0,
    )
    row = i * _B + lax.broadcasted_iota(jnp.int32, (_B, 1), 0)
    h = jnp.where(row < _N, h, 0.0)
    acc[...] += jnp.sum(h, axis=0, keepdims=True)

    @pl.when(i == _GRID - 1)
    def _():
        g = acc[...] * (1.0 / _N)
        out_ref[...] = jnp.dot(g, wl2_ref[...],
                               preferred_element_type=jnp.float32) + bl2_ref[...]


def _tc_pre(x_pad, W1, degp):
    return pl.pallas_call(
        _tc_pre_body,
        grid=(_GRID,),
        in_specs=[
            pl.BlockSpec((_B, _D), lambda i: (i, 0)),
            pl.BlockSpec((_D, _D), lambda i: (0, 0)),
            pl.BlockSpec((_NW, _B), lambda i: (0, i)),
        ],
        out_specs=pl.BlockSpec((_B, _D), lambda i: (i, 0)),
        out_shape=jax.ShapeDtypeStruct((_NP, _D), jnp.float32),
    )(x_pad, W1, degp)


def _tc_mid(aggp, degp, b1r):
    return pl.pallas_call(
        _tc_mid_body,
        grid=(_GRID,),
        in_specs=[
            pl.BlockSpec((_NC, _B, _D), lambda i: (0, i, 0)),
            pl.BlockSpec((_NW, _B), lambda i: (0, i)),
            pl.BlockSpec((1, _D), lambda i: (0, 0)),
        ],
        out_specs=pl.BlockSpec((_B, _D), lambda i: (i, 0)),
        out_shape=jax.ShapeDtypeStruct((_NP, _D), jnp.float32),
    )(aggp, degp, b1r)


def _tc_final(aggp, degp, W2, Wl1, b2r, bl1r, Wl2, bl2r):
    return pl.pallas_call(
        _tc_final_body,
        grid=(_GRID,),
        in_specs=[
            pl.BlockSpec((_NC, _B, _D), lambda i: (0, i, 0)),
            pl.BlockSpec((_NW, _B), lambda i: (0, i)),
            pl.BlockSpec((_D, _D), lambda i: (0, 0)),
            pl.BlockSpec((_D, _D), lambda i: (0, 0)),
            pl.BlockSpec((1, _D), lambda i: (0, 0)),
            pl.BlockSpec((1, _D), lambda i: (0, 0)),
            pl.BlockSpec((_D, _D), lambda i: (0, 0)),
            pl.BlockSpec((1, _D), lambda i: (0, 0)),
        ],
        out_specs=pl.BlockSpec((1, _D), lambda i: (0, 0)),
        out_shape=jax.ShapeDtypeStruct((1, _D), jnp.float32),
        scratch_shapes=[
            pltpu.VMEM((1, _D), jnp.float32),
            pltpu.VMEM((_D, _D), jnp.float32),
            pltpu.VMEM((1, _D), jnp.float32),
        ],
    )(aggp, degp, W2, Wl1, b2r, bl1r, Wl2, bl2r)


def kernel(x, adj, W1, b1, W2, b2, Wl1, bl1, Wl2, bl2):
    src = adj[0]
    dst = adj[1]
    x_pad = jnp.pad(x, ((0, _NP - _N), (0, 0)))
    degp = _sc_degree(dst)
    xt1 = _tc_pre(x_pad, W1, degp)
    aggp1 = _sc_pass(src, dst, xt1)
    xt2 = _tc_mid(aggp1, degp, b1.reshape(1, _D))
    aggp2 = _sc_pass(src, dst, xt2)
    return _tc_final(aggp2, degp, W2, Wl1, b2.reshape(1, _D),
                     bl1.reshape(1, _D), Wl2, bl2.reshape(1, _D))
